# baseline (speedup 1.0000x reference)
import functools

import jax
import jax.numpy as jnp
from jax.experimental import pallas as pl

B, T, D, E, TOP_K = 2, 2048, 1024, 8, 2
ROWS = B * T
TILE = 256


def _moe_kernel(x_ref, wg_ref, wcat_ref, be_ref, out_ref, gate_ref):
    x = x_ref[...]
    xb = x.astype(jnp.bfloat16)

    wg = wg_ref[...].astype(jnp.bfloat16)
    logits = jax.lax.dot_general(
        xb, wg, (((1,), (1,)), ((), ())),
        preferred_element_type=jnp.float32)

    m = jnp.max(logits, axis=1, keepdims=True)
    eg = jnp.exp(logits - m)
    gate = eg / jnp.sum(eg, axis=1, keepdims=True)

    col = jax.lax.broadcasted_iota(jnp.int32, (TILE, E), 1)
    a1 = jnp.argmax(gate, axis=1)[:, None]
    w1 = jnp.max(gate, axis=1)[:, None]
    masked = jnp.where(col == a1, -jnp.inf, gate)
    a2 = jnp.argmax(masked, axis=1)[:, None]
    w2 = jnp.max(masked, axis=1)[:, None]

    G = jnp.where(col == a1, w1, 0.0) + jnp.where(col == a2, w2, 0.0)
    Gb = G.astype(jnp.bfloat16)

    Z = jnp.concatenate(
        [Gb[:, e:e + 1] * xb for e in range(E)], axis=1)

    acc = jax.lax.dot_general(
        Z, wcat_ref[...], (((1,), (0,)), ((), ())),
        preferred_element_type=jnp.float32)

    bias = jax.lax.dot_general(
        G, be_ref[...], (((1,), (0,)), ((), ())),
        preferred_element_type=jnp.float32)

    out_ref[...] = jnp.maximum(acc + bias, 0.0)
    gate_ref[...] = gate


@jax.jit
def kernel(x, Wg, We, be):
    x2 = x.reshape(ROWS, D)
    wcat = We.transpose(0, 2, 1).reshape(E * D, D).astype(jnp.bfloat16)
    grid = (ROWS // TILE,)
    out, gate = pl.pallas_call(
        _moe_kernel,
        grid=grid,
        in_specs=[
            pl.BlockSpec((TILE, D), lambda i: (i, 0)),
            pl.BlockSpec((E, D), lambda i: (0, 0)),
            pl.BlockSpec((E * D, D), lambda i: (0, 0)),
            pl.BlockSpec((E, D), lambda i: (0, 0)),
        ],
        out_specs=[
            pl.BlockSpec((TILE, D), lambda i: (i, 0)),
            pl.BlockSpec((TILE, E), lambda i: (i, 0)),
        ],
        out_shape=[
            jax.ShapeDtypeStruct((ROWS, D), jnp.float32),
            jax.ShapeDtypeStruct((ROWS, E), jnp.float32),
        ],
    )(x2, Wg, wcat, be)
    return out.reshape(B, T, D), gate.reshape(B, T, E)


# speedup vs baseline: 1.3854x; 1.3854x over previous
import functools

import jax
import jax.numpy as jnp
from jax.experimental import pallas as pl

B, T, D, E, TOP_K = 2, 2048, 1024, 8, 2
ROWS = B * T
TILE = 512


def _moe_kernel(x_ref, wg_ref, we_ref, be_ref, out_ref, gate_ref):
    x = x_ref[...]
    xb = x.astype(jnp.bfloat16)

    wg = wg_ref[...].astype(jnp.bfloat16)
    logits = jax.lax.dot_general(
        xb, wg, (((1,), (1,)), ((), ())),
        preferred_element_type=jnp.float32)

    m = jnp.max(logits, axis=1, keepdims=True)
    eg = jnp.exp(logits - m)
    gate = eg / jnp.sum(eg, axis=1, keepdims=True)

    col = jax.lax.broadcasted_iota(jnp.int32, (TILE, E), 1)
    a1 = jnp.argmax(gate, axis=1)[:, None]
    w1 = jnp.max(gate, axis=1)[:, None]
    masked = jnp.where(col == a1, -jnp.inf, gate)
    a2 = jnp.argmax(masked, axis=1)[:, None]
    w2 = jnp.max(masked, axis=1)[:, None]

    G = jnp.where(col == a1, w1, 0.0) + jnp.where(col == a2, w2, 0.0)

    acc = jax.lax.dot_general(
        G, be_ref[...], (((1,), (0,)), ((), ())),
        preferred_element_type=jnp.float32)

    for i in range(E):
        y = jax.lax.dot_general(
            xb, we_ref[i], (((1,), (1,)), ((), ())),
            preferred_element_type=jnp.float32)
        acc = acc + G[:, i:i + 1] * y

    out_ref[...] = jnp.maximum(acc, 0.0)
    gate_ref[...] = gate


@jax.jit
def kernel(x, Wg, We, be):
    x2 = x.reshape(ROWS, D)
    we_bf16 = We.astype(jnp.bfloat16)
    grid = (ROWS // TILE,)
    out, gate = pl.pallas_call(
        _moe_kernel,
        grid=grid,
        in_specs=[
            pl.BlockSpec((TILE, D), lambda i: (i, 0)),
            pl.BlockSpec((E, D), lambda i: (0, 0)),
            pl.BlockSpec((E, D, D), lambda i: (0, 0, 0)),
            pl.BlockSpec((E, D), lambda i: (0, 0)),
        ],
        out_specs=[
            pl.BlockSpec((TILE, D), lambda i: (i, 0)),
            pl.BlockSpec((TILE, E), lambda i: (i, 0)),
        ],
        out_shape=[
            jax.ShapeDtypeStruct((ROWS, D), jnp.float32),
            jax.ShapeDtypeStruct((ROWS, E), jnp.float32),
        ],
    )(x2, Wg, we_bf16, be)
    return out.reshape(B, T, D), gate.reshape(B, T, E)


# speedup vs baseline: 1.3940x; 1.0062x over previous
import functools

import jax
import jax.numpy as jnp
from jax.experimental import pallas as pl

B, T, D, E, TOP_K = 2, 2048, 1024, 8, 2
ROWS = B * T
TILE = 1024


def _moe_kernel(x_ref, wg_ref, we_ref, be_ref, out_ref, gate_ref):
    x = x_ref[...]
    xb = x.astype(jnp.bfloat16)

    wg = wg_ref[...].astype(jnp.bfloat16)
    logits = jax.lax.dot_general(
        xb, wg, (((1,), (1,)), ((), ())),
        preferred_element_type=jnp.float32)

    m = jnp.max(logits, axis=1, keepdims=True)
    eg = jnp.exp(logits - m)
    gate = eg / jnp.sum(eg, axis=1, keepdims=True)

    col = jax.lax.broadcasted_iota(jnp.int32, (TILE, E), 1)
    a1 = jnp.argmax(gate, axis=1)[:, None]
    w1 = jnp.max(gate, axis=1)[:, None]
    masked = jnp.where(col == a1, -jnp.inf, gate)
    a2 = jnp.argmax(masked, axis=1)[:, None]
    w2 = jnp.max(masked, axis=1)[:, None]

    G = jnp.where(col == a1, w1, 0.0) + jnp.where(col == a2, w2, 0.0)

    acc = jax.lax.dot_general(
        G, be_ref[...], (((1,), (0,)), ((), ())),
        preferred_element_type=jnp.float32)

    for i in range(E):
        y = jax.lax.dot_general(
            xb, we_ref[i], (((1,), (1,)), ((), ())),
            preferred_element_type=jnp.float32)
        acc = acc + G[:, i:i + 1] * y

    out_ref[...] = jnp.maximum(acc, 0.0)
    gate_ref[...] = gate


@jax.jit
def kernel(x, Wg, We, be):
    x2 = x.reshape(ROWS, D)
    we_bf16 = We.astype(jnp.bfloat16)
    grid = (ROWS // TILE,)
    out, gate = pl.pallas_call(
        _moe_kernel,
        grid=grid,
        in_specs=[
            pl.BlockSpec((TILE, D), lambda i: (i, 0)),
            pl.BlockSpec((E, D), lambda i: (0, 0)),
            pl.BlockSpec((E, D, D), lambda i: (0, 0, 0)),
            pl.BlockSpec((E, D), lambda i: (0, 0)),
        ],
        out_specs=[
            pl.BlockSpec((TILE, D), lambda i: (i, 0)),
            pl.BlockSpec((TILE, E), lambda i: (i, 0)),
        ],
        out_shape=[
            jax.ShapeDtypeStruct((ROWS, D), jnp.float32),
            jax.ShapeDtypeStruct((ROWS, E), jnp.float32),
        ],
    )(x2, Wg, we_bf16, be)
    return out.reshape(B, T, D), gate.reshape(B, T, E)
